# Initial kernel scaffold; baseline (speedup 1.0000x reference)
import jax
import jax.numpy as jnp
from jax.experimental import pallas as pl


def kernel(x, edge_index, W1, W2):
    raise NotImplementedError("write your pallas kernel here")


# speedup vs baseline: 9.5193x; 9.5193x over previous
import functools
import math

import jax
import jax.numpy as jnp
from jax import lax
from jax.experimental import pallas as pl
from jax.experimental.pallas import tpu as pltpu
from jax.experimental.pallas import tpu_sc as plsc

NC = 2
NS = 16
NW = NC * NS
CHUNK = 128
D = 128
DEGW = 128


def _sc_degree(dst_g, zeros_deg, ones_chunk, npad, kch):
    rpt = npad // NS
    nzc = rpt // CHUNK
    mesh = plsc.VectorSubcoreMesh(core_axis_name="c", subcore_axis_name="s")

    @functools.partial(
        pl.kernel,
        mesh=mesh,
        out_type=jax.ShapeDtypeStruct((NC, npad, DEGW), jnp.float32),
        scratch_types=[
            pltpu.VMEM((1, CHUNK), jnp.int32),
            pltpu.VMEM((1, CHUNK), jnp.int32),
            pltpu.VMEM((CHUNK, DEGW), jnp.float32),
            pltpu.VMEM((CHUNK, DEGW), jnp.float32),
            pltpu.VMEM_SHARED((npad, DEGW), jnp.float32),
            pltpu.SemaphoreType.DMA,
            pltpu.SemaphoreType.DMA,
        ],
    )
    def k(dst_hbm, zeros_hbm, ones_hbm, deg_hbm, ib0, ib1, ones_v, stripe_v,
          acc_sh, sem0, sem1):
        c = lax.axis_index("c")
        s = lax.axis_index("s")
        w = c * NS + s
        pltpu.sync_copy(ones_hbm, ones_v)
        pltpu.sync_copy(zeros_hbm, stripe_v)
        for t in range(nzc):
            pltpu.sync_copy(stripe_v, acc_sh.at[pl.ds(s * rpt + t * CHUNK, CHUNK)])
        plsc.subcore_barrier()

        pltpu.sync_copy(dst_hbm.at[w, pl.ds(0, 1)], ib0)
        pltpu.async_copy(dst_hbm.at[w, pl.ds(1, 1)], ib1, sem1)

        def body(t, carry):
            j2 = jnp.minimum(2 * t + 2, kch - 1)
            j3 = jnp.minimum(2 * t + 3, kch - 1)
            pltpu.sync_copy(ones_v, acc_sh.at[ib0.at[0]], add=True)
            pltpu.make_async_copy(dst_hbm.at[w, pl.ds(j3, 1)], ib1, sem1).wait()
            pltpu.async_copy(dst_hbm.at[w, pl.ds(j2, 1)], ib0, sem0)
            pltpu.sync_copy(ones_v, acc_sh.at[ib1.at[0]], add=True)
            pltpu.make_async_copy(dst_hbm.at[w, pl.ds(j2, 1)], ib0, sem0).wait()
            pltpu.async_copy(dst_hbm.at[w, pl.ds(j3, 1)], ib1, sem1)
            return carry

        lax.fori_loop(0, kch // 2, body, 0)
        pltpu.make_async_copy(dst_hbm.at[w, pl.ds(kch - 1, 1)], ib1, sem1).wait()
        plsc.subcore_barrier()
        for t in range(nzc):
            pltpu.sync_copy(acc_sh.at[pl.ds(s * rpt + t * CHUNK, CHUNK)], stripe_v)
            pltpu.sync_copy(stripe_v, deg_hbm.at[c, pl.ds(s * rpt + t * CHUNK, CHUNK)])

    return k(dst_g, zeros_deg, ones_chunk)


def _sc_aggregate(cmb, table, zeros_chunk, npad, kch):
    rpt = npad // NS
    nzc = rpt // CHUNK
    mesh = plsc.VectorSubcoreMesh(core_axis_name="c", subcore_axis_name="s")

    @functools.partial(
        pl.kernel,
        mesh=mesh,
        out_type=jax.ShapeDtypeStruct((NC, npad, D), jnp.float32),
        scratch_types=[
            pltpu.VMEM((2, CHUNK), jnp.int32),
            pltpu.VMEM((2, CHUNK), jnp.int32),
            pltpu.VMEM((CHUNK, D), jnp.float32),
            pltpu.VMEM((CHUNK, D), jnp.float32),
            pltpu.VMEM_SHARED((npad, D), jnp.float32),
            pltpu.SemaphoreType.DMA,
            pltpu.SemaphoreType.DMA,
            pltpu.SemaphoreType.DMA,
        ],
    )
    def k(cmb_hbm, table_hbm, zeros_hbm, out_hbm,
          iring0, iring1, rows0, rows1, acc_sh, gsem0, gsem1, isem):
        c = lax.axis_index("c")
        s = lax.axis_index("s")
        w = c * NS + s
        pltpu.sync_copy(zeros_hbm, rows0)
        for t in range(nzc):
            pltpu.sync_copy(rows0, acc_sh.at[pl.ds(s * rpt + t * CHUNK, CHUNK)])
        plsc.subcore_barrier()

        pltpu.sync_copy(cmb_hbm.at[w, 0], iring0)
        pltpu.async_copy(table_hbm.at[iring0.at[0]], rows0, gsem0)
        pltpu.async_copy(cmb_hbm.at[w, 1], iring1, isem)

        def body(t, carry):
            j2 = jnp.minimum(2 * t + 2, kch - 1)
            j3 = jnp.minimum(2 * t + 3, kch - 1)
            pltpu.make_async_copy(cmb_hbm.at[w, j3], iring1, isem).wait()
            pltpu.make_async_copy(table_hbm.at[iring0.at[0]], rows0, gsem0).wait()
            pltpu.async_copy(table_hbm.at[iring1.at[0]], rows1, gsem1)
            pltpu.sync_copy(rows0, acc_sh.at[iring0.at[1]], add=True)
            pltpu.sync_copy(cmb_hbm.at[w, j2], iring0)
            pltpu.async_copy(table_hbm.at[iring0.at[0]], rows0, gsem0)
            pltpu.make_async_copy(table_hbm.at[iring1.at[0]], rows1, gsem1).wait()
            pltpu.sync_copy(rows1, acc_sh.at[iring1.at[1]], add=True)
            pltpu.async_copy(cmb_hbm.at[w, j3], iring1, isem)
            return carry

        lax.fori_loop(0, kch // 2, body, 0)
        pltpu.make_async_copy(table_hbm.at[iring0.at[0]], rows0, gsem0).wait()
        pltpu.make_async_copy(cmb_hbm.at[w, kch - 1], iring1, isem).wait()
        plsc.subcore_barrier()
        for t in range(nzc):
            pltpu.sync_copy(acc_sh.at[pl.ds(s * rpt + t * CHUNK, CHUNK)], rows0)
            pltpu.sync_copy(rows0, out_hbm.at[c, pl.ds(s * rpt + t * CHUNK, CHUNK)])

    return k(cmb, table, zeros_chunk)


def _first_body(dp_ref, x_ref, w_ref, t_ref, dinv_ref):
    deg = dp_ref[0, :, 0:1] + dp_ref[1, :, 0:1]
    dinv = jnp.where(deg > 0.0, lax.rsqrt(jnp.maximum(deg, 1.0)), 0.0)
    xw = jnp.dot(x_ref[...], w_ref[...], preferred_element_type=jnp.float32,
                 precision=lax.Precision.HIGHEST)
    t_ref[...] = dinv * xw
    dinv_ref[...] = dinv


def _tc_first(deg_p, x_pad, W1, npad):
    nb = npad // 128
    return pl.pallas_call(
        _first_body,
        grid=(nb,),
        in_specs=[
            pl.BlockSpec((NC, 128, DEGW), lambda j: (0, j, 0)),
            pl.BlockSpec((128, D), lambda j: (j, 0)),
            pl.BlockSpec((D, D), lambda j: (0, 0)),
        ],
        out_specs=[
            pl.BlockSpec((128, D), lambda j: (j, 0)),
            pl.BlockSpec((128, 1), lambda j: (j, 0)),
        ],
        out_shape=[
            jax.ShapeDtypeStruct((npad, D), jnp.float32),
            jax.ShapeDtypeStruct((npad, 1), jnp.float32),
        ],
    )(deg_p, x_pad, W1)


def _mid_body(p_ref, dinv_ref, w_ref, out_ref):
    agg = p_ref[0] + p_ref[1]
    dinv = dinv_ref[...]
    h = jnp.maximum(dinv * agg, 0.0)
    out_ref[...] = dinv * jnp.dot(h, w_ref[...], preferred_element_type=jnp.float32,
                                  precision=lax.Precision.HIGHEST)


def _tc_mid(p, dinv, W2, npad):
    nb = npad // 128
    return pl.pallas_call(
        _mid_body,
        grid=(nb,),
        in_specs=[
            pl.BlockSpec((NC, 128, D), lambda j: (0, j, 0)),
            pl.BlockSpec((128, 1), lambda j: (j, 0)),
            pl.BlockSpec((D, D), lambda j: (0, 0)),
        ],
        out_specs=pl.BlockSpec((128, D), lambda j: (j, 0)),
        out_shape=jax.ShapeDtypeStruct((npad, D), jnp.float32),
    )(p, dinv, W2)


def _final_body(p_ref, dinv_ref, out_ref):
    out_ref[...] = dinv_ref[...] * (p_ref[0] + p_ref[1])


def _tc_final(p, dinv, npad):
    nb = npad // 128
    return pl.pallas_call(
        _final_body,
        grid=(nb,),
        in_specs=[
            pl.BlockSpec((NC, 128, D), lambda j: (0, j, 0)),
            pl.BlockSpec((128, 1), lambda j: (j, 0)),
        ],
        out_specs=pl.BlockSpec((128, D), lambda j: (j, 0)),
        out_shape=jax.ShapeDtypeStruct((npad, D), jnp.float32),
    )(p, dinv)


def kernel(x, edge_index, W1, W2):
    n = x.shape[0]
    e = edge_index.shape[1]
    npad = math.ceil((n + 1) / (NS * CHUNK)) * (NS * CHUNK)
    kch = 2 * math.ceil(e / (NW * CHUNK * 2))
    e_pad = NW * CHUNK * kch
    pad_region = npad - n
    pad_idx = (jnp.arange(e_pad - e, dtype=jnp.int32) % pad_region) + n
    src_p = jnp.concatenate([edge_index[0], pad_idx])
    dst_p = jnp.concatenate([edge_index[1], pad_idx])
    g_total = NW * kch
    order = jnp.argsort(dst_p)
    src_c = src_p[order].reshape(CHUNK, g_total).T
    dst_c = dst_p[order].reshape(CHUNK, g_total).T
    src_g = src_c.reshape(kch, NW, CHUNK).transpose(1, 0, 2)
    dst_g = dst_c.reshape(kch, NW, CHUNK).transpose(1, 0, 2)
    cmb = jnp.stack([src_g, dst_g], axis=2)
    zeros_stripe = jnp.zeros((CHUNK, DEGW), jnp.float32)
    ones_chunk = jnp.ones((CHUNK, DEGW), jnp.float32)
    zeros_chunk = jnp.zeros((CHUNK, D), jnp.float32)
    x_pad = jnp.pad(x, ((0, npad - n), (0, 0)))

    deg_p = _sc_degree(dst_g, zeros_stripe, ones_chunk, npad, kch)
    t1s, dinv = _tc_first(deg_p, x_pad, W1, npad)
    p1 = _sc_aggregate(cmb, t1s, zeros_chunk, npad, kch)
    t2s = _tc_mid(p1, dinv, W2, npad)
    p2 = _sc_aggregate(cmb, t2s, zeros_chunk, npad, kch)
    out = _tc_final(p2, dinv, npad)
    return out[:n]
